# Initial kernel scaffold; baseline (speedup 1.0000x reference)
import jax
import jax.numpy as jnp
from jax.experimental import pallas as pl


def kernel(x, edge_index, W_msg, W_self, W_upd, b):
    raise NotImplementedError("write your pallas kernel here")


# speedup vs baseline: 7.0851x; 7.0851x over previous
import functools

import jax
import jax.numpy as jnp
from jax import lax
from jax.experimental import pallas as pl
from jax.experimental.pallas import tpu as pltpu
from jax.experimental.pallas import tpu_sc as plsc

N = 10000
E = 320000
D = 128
NL = 3

NC = 2
NS = 16
NW = NC * NS

E_PER_TILE = E // NW
CHUNK = 80
N_CHUNKS = E_PER_TILE // CHUNK
STRIPE = 624
TAIL0 = NS * STRIPE
TAIL = N - TAIL0


def _sc_partial_segsum(h, src_r, dst_r, z):
  mesh = plsc.VectorSubcoreMesh(core_axis_name="c", subcore_axis_name="s")

  @functools.partial(
      pl.kernel,
      out_type=jax.ShapeDtypeStruct((NC, N, D), jnp.float32),
      mesh=mesh,
      scratch_types=[
          pltpu.VMEM((N_CHUNKS, CHUNK), jnp.int32),
          pltpu.VMEM((N_CHUNKS, CHUNK), jnp.int32),
          pltpu.VMEM((CHUNK, D), jnp.float32),
          pltpu.VMEM_SHARED((N, D), jnp.float32),
          pltpu.SemaphoreType.DMA,
      ],
  )
  def k(h_hbm, src_hbm, dst_hbm, z_hbm, out_hbm, src_v, dst_v, rows_v, acc_sh, sem):
    cid = lax.axis_index("c")
    sid = lax.axis_index("s")
    wid = cid * NS + sid
    row0 = sid * STRIPE
    pltpu.sync_copy(z_hbm.at[pl.ds(row0, STRIPE)],
                    acc_sh.at[pl.ds(row0, STRIPE)])

    @pl.when(sid == NS - 1)
    def _():
      pltpu.sync_copy(z_hbm.at[pl.ds(TAIL0, TAIL)], acc_sh.at[pl.ds(TAIL0, TAIL)])

    pltpu.sync_copy(src_hbm.at[wid], src_v)
    pltpu.sync_copy(dst_hbm.at[wid], dst_v)
    plsc.subcore_barrier()

    @pl.loop(0, N_CHUNKS)
    def _(c):
      pltpu.async_copy(h_hbm.at[src_v.at[c]], rows_v, sem).wait()
      pltpu.sync_copy(rows_v, acc_sh.at[dst_v.at[c]], add=True)

    plsc.subcore_barrier()
    pltpu.sync_copy(acc_sh.at[pl.ds(row0, STRIPE)],
                    out_hbm.at[cid].at[pl.ds(row0, STRIPE)])

    @pl.when(sid == NS - 1)
    def _():
      pltpu.sync_copy(acc_sh.at[pl.ds(TAIL0, TAIL)],
                      out_hbm.at[cid].at[pl.ds(TAIL0, TAIL)])

  return k(h, src_r, dst_r, z)


def _tc_update(h, A, Wm, Ws, Wu, bias):
  BLK = 1000

  def body(h_ref, a0_ref, a1_ref, wm_ref, ws_ref, wu_ref, b_ref, o_ref):
    a = a0_ref[...] + a1_ref[...]
    agg = jnp.dot(a, wm_ref[...], preferred_element_type=jnp.float32)
    out = (jnp.dot(h_ref[...], ws_ref[...], preferred_element_type=jnp.float32)
           + jnp.dot(agg, wu_ref[...], preferred_element_type=jnp.float32)
           + b_ref[...])
    o_ref[...] = jnp.maximum(out, 0.0)

  return pl.pallas_call(
      body,
      grid=(N // BLK,),
      in_specs=[
          pl.BlockSpec((BLK, D), lambda i: (i, 0)),
          pl.BlockSpec((BLK, D), lambda i: (i, 0)),
          pl.BlockSpec((BLK, D), lambda i: (i, 0)),
          pl.BlockSpec((D, D), lambda i: (0, 0)),
          pl.BlockSpec((D, D), lambda i: (0, 0)),
          pl.BlockSpec((D, D), lambda i: (0, 0)),
          pl.BlockSpec((1, D), lambda i: (0, 0)),
      ],
      out_specs=pl.BlockSpec((BLK, D), lambda i: (i, 0)),
      out_shape=jax.ShapeDtypeStruct((N, D), jnp.float32),
  )(h, A[0], A[1], Wm, Ws, Wu, bias)


def kernel(x, edge_index, W_msg, W_self, W_upd, b):
  src = edge_index[0].astype(jnp.int32).reshape(NW, N_CHUNKS, CHUNK)
  dst = edge_index[1].astype(jnp.int32).reshape(NW, N_CHUNKS, CHUNK)
  z = jnp.zeros((N, D), jnp.float32)
  bias = b.reshape(NL, 1, D)
  h = x
  for l in range(NL):
    A = _sc_partial_segsum(h, src, dst, z)
    h = _tc_update(h, A, W_msg[l], W_self[l], W_upd[l], bias[l])
  return h
